# baseline (speedup 1.0000x reference)
import functools

import jax
import jax.numpy as jnp
from jax import lax
from jax.experimental import pallas as pl
from jax.experimental.pallas import tpu as pltpu
from jax.experimental.pallas import tpu_sc as plsc

NC, NS, L = 2, 16, 16
NW = NC * NS
SEG = 128
K = 6
TB = K * SEG // 3
RPB = 3 * TB


def _make_sc_call(n_triples, vocab, embed):
    assert embed == 2 * L
    segs = 3 * n_triples // SEG
    assert segs % (NW * K) in (0, NW * (K // 2))
    blocks = segs // K
    assert blocks % NW == 0
    bpw = blocks // NW
    spw = segs // NW

    mesh = plsc.VectorSubcoreMesh(core_axis_name="c", subcore_axis_name="s")

    @functools.partial(
        pl.kernel,
        out_type=jax.ShapeDtypeStruct((n_triples, embed), jnp.float32),
        mesh=mesh,
        scratch_types=[
            pltpu.VMEM((spw, SEG), jnp.int32),
            pltpu.VMEM((2, RPB, embed), jnp.float32),
            pltpu.VMEM((TB, embed), jnp.float32),
            pltpu.VMEM((16, L), jnp.float32),
            pltpu.SemaphoreType.DMA,
            pltpu.SemaphoreType.DMA,
        ],
        compiler_params=pltpu.CompilerParams(use_tc_tiling_on_sc=False),
    )
    def sc_call(idx_hbm, table_hbm, w_hbm, out_hbm, idx_v, rows_v, out_v, wv,
                gsem0, gsem1):
        wid = lax.axis_index("s") * NC + lax.axis_index("c")
        pltpu.sync_copy(w_hbm, wv)
        pltpu.sync_copy(idx_hbm.at[pl.ds(wid * spw, spw)], idx_v)
        gsems = (gsem0, gsem1)

        a = [[wv[3 * q + p] for p in range(3)] for q in range(3)]
        b0v = [wv[9 + p] for p in range(3)]
        w1v = [wv[12 + p] for p in range(3)]
        b1v = wv[15]

        def gather_descs(buf, blk):
            return [
                pltpu.make_async_copy(
                    table_hbm.at[idx_v.at[blk * K + k]],
                    rows_v.at[buf].at[pl.ds(k * SEG, SEG)],
                    gsems[buf],
                )
                for k in range(K)
            ]

        def issue(buf, blk):
            for cp in gather_descs(buf, blk):
                cp.start()

        def drain(buf, blk):
            for cp in gather_descs(buf, blk):
                cp.wait()

        def process(buf, blk):
            rb = rows_v.at[buf]

            @plsc.parallel_loop(0, TB, unroll=4)
            def _(t):
                r = 3 * t
                for v in range(2):
                    sl = pl.ds(v * L, L)
                    e0 = rb[r, sl]
                    e1 = rb[r + 1, sl]
                    e2 = rb[r + 2, sl]
                    o = b1v
                    for p in range(3):
                        h = e0 * a[0][p] + e1 * a[1][p] + e2 * a[2][p] + b0v[p]
                        h = jnp.maximum(h, 0.0)
                        o = o + h * w1v[p]
                    out_v[t, sl] = jnp.maximum(o, 0.0)

            g = wid * bpw + blk
            pltpu.sync_copy(out_v, out_hbm.at[pl.ds(g * TB, TB)])

        issue(0, 0)

        def pair_body(j, carry):
            b0i = 2 * j

            @pl.when(b0i + 1 < bpw)
            def _():
                issue(1, b0i + 1)

            drain(0, b0i)
            process(0, b0i)

            @pl.when(b0i + 2 < bpw)
            def _():
                issue(0, b0i + 2)

            @pl.when(b0i + 1 < bpw)
            def _():
                drain(1, b0i + 1)
                process(1, b0i + 1)

            return carry

        lax.fori_loop(0, (bpw + 1) // 2, pair_body, 0)

    return sc_call


def kernel(inputs, table, W0, b0, W1, b1):
    B, S, M, P = inputs.shape
    vocab, embed = table.shape
    assert P == 3
    n = B * S * M
    idx3 = inputs.reshape(-1).astype(jnp.int32).reshape(3 * n // SEG, SEG)

    A = W0 + jnp.eye(P, dtype=W0.dtype)
    wflat = jnp.concatenate([A.reshape(-1), b0, W1.reshape(-1), b1])
    wvec = jnp.broadcast_to(wflat[:, None], (16, L)).astype(jnp.float32)

    out = _make_sc_call(n, vocab, embed)(idx3, table, wvec)
    return out.reshape(B, S, M, embed)
